# baseline (speedup 1.0000x reference)
import functools

import jax
import jax.numpy as jnp
from jax.experimental import pallas as pl
from jax.experimental.pallas import tpu as pltpu

_CUT = 5.0
_PP = 5
_SBF = 7
_RBF = 16
_BE = 2048
_BT = 2048


def _silu(v):
    return v * jax.nn.sigmoid(v)


def _env(d):
    xx = d / _CUT
    p = _PP
    e = (1.0 - ((p + 1) * (p + 2) / 2.0) * xx ** p
         + p * (p + 2) * xx ** (p + 1)
         - (p * (p + 1) / 2.0) * xx ** (p + 2))
    return jnp.where(d < _CUT, e, 0.0)


def _edge_body(n_real, dvec_ref, eattr_ref, eae_ref, wmat_ref, bmat_ref,
               wemb_ref, bemb_ref, wrbf_ref, brbf_ref, wea0_ref, wea1_ref,
               h0_ref, rb_ref, ea0_ref, ea1_ref, racc_ref):
    i = pl.program_id(0)
    blk = dvec_ref.shape[0]
    pos = i * blk + jax.lax.broadcasted_iota(jnp.int32, (blk, 1), 0)
    valid = pos < n_real
    dv = dvec_ref[...]
    d = jnp.sqrt(jnp.sum(dv * dv, axis=1, keepdims=True) + 1e-12)
    env = _env(d)
    ea = eattr_ref[...] * env
    t1 = _silu(jnp.dot(ea, wmat_ref[...], preferred_element_type=jnp.float32)
               + bmat_ref[...])
    h0_ref[...] = _silu(jnp.dot(t1, wemb_ref[...],
                                preferred_element_type=jnp.float32)
                        + bemb_ref[...])
    n = (1 + jax.lax.broadcasted_iota(jnp.int32, (blk, _RBF), 1)
         ).astype(jnp.float32)
    rb = (jnp.sqrt(2.0 / _CUT) * jnp.sin(n * (jnp.pi / _CUT) * d)
          / (d + 1e-9)) * env
    rb_ref[...] = rb
    eae = eae_ref[...]
    ea0_ref[...] = _silu(jnp.dot(eae, wea0_ref[...],
                                 preferred_element_type=jnp.float32))
    ea1_ref[...] = _silu(jnp.dot(eae, wea1_ref[...],
                                 preferred_element_type=jnp.float32))
    rbf64 = _silu(jnp.dot(rb, wrbf_ref[...],
                          preferred_element_type=jnp.float32) + brbf_ref[...])
    rbf64 = jnp.where(valid, rbf64, 0.0)
    colsum = jnp.sum(rbf64, axis=0, keepdims=True)

    @pl.when(i == 0)
    def _():
        racc_ref[...] = jnp.zeros_like(racc_ref)

    row0 = jax.lax.broadcasted_iota(jnp.int32, (8, 64), 0) == 0
    racc_ref[...] += jnp.where(row0, jnp.broadcast_to(colsum, (8, 64)), 0.0)


def _triplet_body(n_real, dv1_ref, dv2_ref, rbg_ref, hg_ref, eag_ref,
                  wsbf_ref, msg_ref):
    i = pl.program_id(0)
    blk = dv1_ref.shape[0]
    pos = i * blk + jax.lax.broadcasted_iota(jnp.int32, (blk, 1), 0)
    dv1 = dv1_ref[...]
    dv2 = dv2_ref[...]
    cos_a = -jnp.sum(dv1 * dv2, axis=1, keepdims=True)
    c0 = dv1[:, 1:2] * dv2[:, 2:3] - dv1[:, 2:3] * dv2[:, 1:2]
    c1 = dv1[:, 2:3] * dv2[:, 0:1] - dv1[:, 0:1] * dv2[:, 2:3]
    c2 = dv1[:, 0:1] * dv2[:, 1:2] - dv1[:, 1:2] * dv2[:, 0:1]
    sin2 = c0 * c0 + c1 * c1 + c2 * c2
    hyp = jnp.sqrt(cos_a * cos_a + sin2)
    cost = jnp.where(hyp > 1e-30, cos_a / jnp.maximum(hyp, 1e-30), 1.0)
    rbg = rbg_ref[...]
    angs = [jnp.ones_like(cost), cost]
    for _ in range(2, _SBF):
        angs.append(2.0 * cost * angs[-1] - angs[-2])
    sbf = jnp.concatenate([a * rbg for a in angs], axis=1)
    gate = _silu(jnp.dot(sbf, wsbf_ref[...],
                         preferred_element_type=jnp.float32))
    msg = hg_ref[...] * gate + eag_ref[...]
    msg_ref[...] = jnp.where(pos < n_real, msg, 0.0)


def _update_body(h_ref, agg_ref, wlin_ref, blin_ref, out_ref):
    out_ref[...] = _silu(
        jnp.dot(h_ref[...] + agg_ref[...], wlin_ref[...],
                preferred_element_type=jnp.float32) + blin_ref[...])


def _final_body(n_real, nblk, h_ref, isb_ref, racc_ref, wr2_ref, wout_ref,
                bout_ref, out_ref, acc_ref):
    i = pl.program_id(0)
    blk = h_ref.shape[0]
    pos = i * blk + jax.lax.broadcasted_iota(jnp.int32, (blk, 1), 0)
    valid = pos < n_real
    wb = jnp.where(isb_ref[...] > 0.5, 1.0, 0.5)
    hw = jnp.where(valid, h_ref[...] * wb, 0.0)
    colsum = jnp.sum(hw, axis=0, keepdims=True)

    @pl.when(i == 0)
    def _():
        acc_ref[...] = jnp.zeros_like(acc_ref)

    row0 = jax.lax.broadcasted_iota(jnp.int32, (8, 128), 0) == 0
    acc_ref[...] += jnp.where(row0, jnp.broadcast_to(colsum, (8, 128)), 0.0)

    @pl.when(i == nblk - 1)
    def _():
        rsum = racc_ref[0:1, :]
        total = acc_ref[0:1, :] + jnp.dot(
            rsum, wr2_ref[...], preferred_element_type=jnp.float32)
        out_ref[...] = (jnp.dot(total, wout_ref[...],
                                preferred_element_type=jnp.float32)
                        + bout_ref[...])


def _pad_rows(a, n):
    if a.shape[0] == n:
        return a
    return jnp.pad(a, [(0, n - a.shape[0])] + [(0, 0)] * (a.ndim - 1))


def kernel(atom_pos, edge_attr, x, edge_index, is_bond, neo_edge_index,
           atom_j, atom_i, atom_k, emb_table, W_mat, b_mat, W_rbf, b_rbf,
           W_emb, b_emb, Wsbf, Wea, Wlin, blin, Wr2, Wout, bout):
    f32 = jnp.float32
    src = edge_index[0].astype(jnp.int32)
    dst = edge_index[1].astype(jnp.int32)
    e1 = neo_edge_index[0].astype(jnp.int32)
    e2 = neo_edge_index[1].astype(jnp.int32)
    E = edge_attr.shape[0]
    T = e1.shape[0]
    Ep = ((E + _BE - 1) // _BE) * _BE
    Tp = ((T + _BT - 1) // _BT) * _BT
    ge = Ep // _BE
    gt = Tp // _BT

    dvec = atom_pos[src] - atom_pos[dst]
    eae = emb_table[x][dst]

    dvec_p = _pad_rows(dvec, Ep)
    eattr_p = _pad_rows(edge_attr, Ep)
    eae_p = _pad_rows(eae, Ep)

    wfull = lambda shp: pl.BlockSpec(shp, lambda i: (0, 0))
    eblk = lambda c: pl.BlockSpec((_BE, c), lambda i: (i, 0))
    tblk = lambda c: pl.BlockSpec((_BT, c), lambda i: (i, 0))

    h0, rbenv, ea0, ea1, racc = pl.pallas_call(
        functools.partial(_edge_body, E),
        grid=(ge,),
        in_specs=[eblk(3), eblk(338), eblk(64), wfull((338, 128)),
                  wfull((1, 128)), wfull((128, 128)), wfull((1, 128)),
                  wfull((16, 64)), wfull((1, 64)), wfull((64, 128)),
                  wfull((64, 128))],
        out_specs=[eblk(128), eblk(16), eblk(128), eblk(128),
                   pl.BlockSpec((8, 64), lambda i: (0, 0))],
        out_shape=[jax.ShapeDtypeStruct((Ep, 128), f32),
                   jax.ShapeDtypeStruct((Ep, 16), f32),
                   jax.ShapeDtypeStruct((Ep, 128), f32),
                   jax.ShapeDtypeStruct((Ep, 128), f32),
                   jax.ShapeDtypeStruct((8, 64), f32)],
    )(dvec_p, eattr_p, eae_p, W_mat, b_mat.reshape(1, 128),
      W_emb, b_emb.reshape(1, 128), W_rbf, b_rbf.reshape(1, 64),
      Wea[0], Wea[1])

    dv1 = _pad_rows(dvec[e1], Tp)
    dv2 = _pad_rows(dvec[e2], Tp)
    rbg = _pad_rows(rbenv[:E][e1] if Ep != E else rbenv[e1], Tp)
    e2p = _pad_rows(e2[:, None], Tp)[:, 0]

    h = h0
    eas = (ea0, ea1)
    for l in range(Wlin.shape[0]):
        hg = _pad_rows(h[e1], Tp)
        eag = _pad_rows(eas[l][e1], Tp)
        msg = pl.pallas_call(
            functools.partial(_triplet_body, T),
            grid=(gt,),
            in_specs=[tblk(3), tblk(3), tblk(16), tblk(128), tblk(128),
                      wfull((112, 128))],
            out_specs=tblk(128),
            out_shape=jax.ShapeDtypeStruct((Tp, 128), f32),
        )(dv1, dv2, rbg, hg, eag, Wsbf[l])
        agg = jax.ops.segment_sum(msg, e2p, num_segments=E)
        agg_p = _pad_rows(agg, Ep)
        h = pl.pallas_call(
            _update_body,
            grid=(ge,),
            in_specs=[eblk(128), eblk(128), wfull((128, 128)),
                      wfull((1, 128))],
            out_specs=eblk(128),
            out_shape=jax.ShapeDtypeStruct((Ep, 128), f32),
        )(h, agg_p, Wlin[l], blin[l].reshape(1, 128))

    isb_p = _pad_rows(is_bond.astype(f32)[:, None], Ep)
    out = pl.pallas_call(
        functools.partial(_final_body, E, ge),
        grid=(ge,),
        in_specs=[eblk(128), eblk(1), pl.BlockSpec((8, 64), lambda i: (0, 0)),
                  wfull((64, 128)), wfull((128, 1)), wfull((1, 1))],
        out_specs=pl.BlockSpec((1, 1), lambda i: (0, 0)),
        out_shape=jax.ShapeDtypeStruct((1, 1), f32),
        scratch_shapes=[pltpu.VMEM((8, 128), f32)],
    )(h, isb_p, racc, Wr2, Wout, bout.reshape(1, 1))
    return out


# speedup vs baseline: 7.8394x; 7.8394x over previous
import functools

import jax
import jax.numpy as jnp
from jax import lax
from jax.experimental import pallas as pl
from jax.experimental.pallas import tpu as pltpu
from jax.experimental.pallas import tpu_sc as plsc

_CUT = 5.0
_PP = 5
_SBF = 7
_RBF = 16
_BE = 1600
_BT = 2048
_NW = 32
_CH = 128


def _silu(v):
    return v * jax.nn.sigmoid(v)


def _env(d):
    xx = d / _CUT
    p = _PP
    e = (1.0 - ((p + 1) * (p + 2) / 2.0) * xx ** p
         + p * (p + 2) * xx ** (p + 1)
         - (p * (p + 1) / 2.0) * xx ** (p + 2))
    return jnp.where(d < _CUT, e, 0.0)


@functools.lru_cache(maxsize=None)
def _gather_fn(D, M):
    per_w = M // _NW
    n_ch = per_w // _CH
    mesh = plsc.VectorSubcoreMesh(core_axis_name="c", subcore_axis_name="s")

    @functools.partial(
        pl.kernel, mesh=mesh,
        out_type=jax.ShapeDtypeStruct((M, D), jnp.float32),
        scratch_types=[pltpu.VMEM((_CH,), jnp.int32),
                       pltpu.VMEM((_CH, D), jnp.float32),
                       pltpu.SemaphoreType.DMA],
    )
    def k(table_hbm, idx_hbm, out_hbm, idx_v, rows_v, sem):
        wid = lax.axis_index("s") * 2 + lax.axis_index("c")
        base = wid * per_w

        def body(i, carry):
            st = pl.multiple_of(base + i * _CH, _CH)
            pltpu.sync_copy(idx_hbm.at[pl.ds(st, _CH)], idx_v)
            pltpu.async_copy(table_hbm.at[idx_v], rows_v, sem).wait()
            pltpu.sync_copy(rows_v, out_hbm.at[pl.ds(st, _CH)])
            return carry

        lax.fori_loop(0, n_ch, body, 0)

    return k


def _sc_gather(table, idx):
    return _gather_fn(table.shape[1], idx.shape[0])(table, idx)


def _edge_body(dv_ref, eattr_ref, eae_ref, wmat_ref, bmat_ref,
               wemb_ref, bemb_ref, wrbf_ref, brbf_ref, wea0_ref, wea1_ref,
               he0_ref, geom_ref, ea1_ref, racc_ref):
    i = pl.program_id(0)
    blk = dv_ref.shape[0]
    dv = dv_ref[...]
    d = jnp.sqrt(jnp.sum(dv * dv, axis=1, keepdims=True) + 1e-12)
    env = _env(d)
    ea = eattr_ref[...] * env
    t1 = _silu(jnp.dot(ea, wmat_ref[...], preferred_element_type=jnp.float32)
               + bmat_ref[...])
    h0 = _silu(jnp.dot(t1, wemb_ref[...], preferred_element_type=jnp.float32)
               + bemb_ref[...])
    n = (1 + jax.lax.broadcasted_iota(jnp.int32, (blk, _RBF), 1)
         ).astype(jnp.float32)
    rb = (jnp.sqrt(2.0 / _CUT) * jnp.sin(n * (jnp.pi / _CUT) * d)
          / (d + 1e-9)) * env
    geom_ref[...] = jnp.concatenate(
        [dv, rb, jnp.zeros((blk, 128 - 19), jnp.float32)], axis=1)
    eae = eae_ref[...]
    ea0 = _silu(jnp.dot(eae, wea0_ref[...],
                        preferred_element_type=jnp.float32))
    ea1_ref[...] = _silu(jnp.dot(eae, wea1_ref[...],
                                 preferred_element_type=jnp.float32))
    he0_ref[...] = jnp.concatenate([h0, ea0], axis=1)
    rbf64 = _silu(jnp.dot(rb, wrbf_ref[...],
                          preferred_element_type=jnp.float32) + brbf_ref[...])
    colsum = jnp.sum(rbf64, axis=0, keepdims=True)

    @pl.when(i == 0)
    def _():
        racc_ref[...] = jnp.zeros_like(racc_ref)

    row0 = jax.lax.broadcasted_iota(jnp.int32, (8, 64), 0) == 0
    racc_ref[...] += jnp.where(row0, jnp.broadcast_to(colsum, (8, 64)), 0.0)


def _triplet_body(n_real, g1_ref, g2_ref, he_ref, wsbf_ref, msg_ref):
    i = pl.program_id(0)
    blk = g1_ref.shape[0]
    pos = i * blk + jax.lax.broadcasted_iota(jnp.int32, (blk, 1), 0)
    dv1 = g1_ref[:, 0:3]
    dv2 = g2_ref[:, 0:3]
    rbg = g1_ref[:, 3:19]
    cos_a = -jnp.sum(dv1 * dv2, axis=1, keepdims=True)
    c0 = dv1[:, 1:2] * dv2[:, 2:3] - dv1[:, 2:3] * dv2[:, 1:2]
    c1 = dv1[:, 2:3] * dv2[:, 0:1] - dv1[:, 0:1] * dv2[:, 2:3]
    c2 = dv1[:, 0:1] * dv2[:, 1:2] - dv1[:, 1:2] * dv2[:, 0:1]
    sin2 = c0 * c0 + c1 * c1 + c2 * c2
    hyp = jnp.sqrt(cos_a * cos_a + sin2)
    cost = jnp.where(hyp > 1e-30, cos_a / jnp.maximum(hyp, 1e-30), 1.0)
    angs = [jnp.ones_like(cost), cost]
    for _ in range(2, _SBF):
        angs.append(2.0 * cost * angs[-1] - angs[-2])
    sbf = jnp.concatenate([a * rbg for a in angs], axis=1)
    gate = _silu(jnp.dot(sbf, wsbf_ref[...],
                         preferred_element_type=jnp.float32))
    msg = he_ref[:, 0:128] * gate + he_ref[:, 128:256]
    msg_ref[...] = jnp.where(pos < n_real, msg, 0.0)


def _update0_body(he_ref, agg_ref, ea1_ref, wlin_ref, blin_ref, out_ref):
    h1 = _silu(jnp.dot(he_ref[:, 0:128] + agg_ref[...], wlin_ref[...],
                       preferred_element_type=jnp.float32) + blin_ref[...])
    out_ref[...] = jnp.concatenate([h1, ea1_ref[...]], axis=1)


def _update1_body(he_ref, agg_ref, wlin_ref, blin_ref, out_ref):
    out_ref[...] = _silu(
        jnp.dot(he_ref[:, 0:128] + agg_ref[...], wlin_ref[...],
                preferred_element_type=jnp.float32) + blin_ref[...])


def _final_body(nblk, h_ref, isb_ref, racc_ref, wr2_ref, wout_ref,
                bout_ref, out_ref, acc_ref):
    i = pl.program_id(0)
    wb = jnp.where(isb_ref[...] > 0.5, 1.0, 0.5)
    hw = h_ref[...] * wb
    colsum = jnp.sum(hw, axis=0, keepdims=True)

    @pl.when(i == 0)
    def _():
        acc_ref[...] = jnp.zeros_like(acc_ref)

    row0 = jax.lax.broadcasted_iota(jnp.int32, (8, 128), 0) == 0
    acc_ref[...] += jnp.where(row0, jnp.broadcast_to(colsum, (8, 128)), 0.0)

    @pl.when(i == nblk - 1)
    def _():
        rsum = racc_ref[0:1, :]
        total = acc_ref[0:1, :] + jnp.dot(
            rsum, wr2_ref[...], preferred_element_type=jnp.float32)
        out_ref[...] = (jnp.dot(total, wout_ref[...],
                                preferred_element_type=jnp.float32)
                        + bout_ref[...])


def _pad_to(a, n):
    return a if a.shape[0] == n else jnp.pad(a, (0, n - a.shape[0]))


def kernel(atom_pos, edge_attr, x, edge_index, is_bond, neo_edge_index,
           atom_j, atom_i, atom_k, emb_table, W_mat, b_mat, W_rbf, b_rbf,
           W_emb, b_emb, Wsbf, Wea, Wlin, blin, Wr2, Wout, bout):
    f32 = jnp.float32
    src = edge_index[0].astype(jnp.int32)
    dst = edge_index[1].astype(jnp.int32)
    e1 = neo_edge_index[0].astype(jnp.int32)
    e2 = neo_edge_index[1].astype(jnp.int32)
    E = edge_attr.shape[0]
    T = e1.shape[0]
    unit = _NW * _CH
    Tp = ((T + unit - 1) // unit) * unit
    ge = E // _BE
    gt = Tp // _BT

    e1p = _pad_to(e1, Tp)
    e2p = _pad_to(e2, Tp)

    dvec = atom_pos[src] - atom_pos[dst]
    eae = emb_table[x][dst]

    wfull = lambda shp: pl.BlockSpec(shp, lambda i: (0, 0))
    eblk = lambda c: pl.BlockSpec((_BE, c), lambda i: (i, 0))
    tblk = lambda c: pl.BlockSpec((_BT, c), lambda i: (i, 0))

    he0, geom, ea1, racc = pl.pallas_call(
        _edge_body,
        grid=(ge,),
        in_specs=[eblk(3), eblk(338), eblk(64), wfull((338, 128)),
                  wfull((1, 128)), wfull((128, 128)), wfull((1, 128)),
                  wfull((16, 64)), wfull((1, 64)), wfull((64, 128)),
                  wfull((64, 128))],
        out_specs=[eblk(256), eblk(128), eblk(128),
                   pl.BlockSpec((8, 64), lambda i: (0, 0))],
        out_shape=[jax.ShapeDtypeStruct((E, 256), f32),
                   jax.ShapeDtypeStruct((E, 128), f32),
                   jax.ShapeDtypeStruct((E, 128), f32),
                   jax.ShapeDtypeStruct((8, 64), f32)],
    )(dvec, edge_attr, eae, W_mat, b_mat.reshape(1, 128),
      W_emb, b_emb.reshape(1, 128), W_rbf, b_rbf.reshape(1, 64),
      Wea[0], Wea[1])

    g1 = _sc_gather(geom, e1p)
    g2 = _sc_gather(geom, e2p)

    he = he0
    for l in range(Wlin.shape[0]):
        heg = _sc_gather(he, e1p)
        msg = pl.pallas_call(
            functools.partial(_triplet_body, T),
            grid=(gt,),
            in_specs=[tblk(128), tblk(128), tblk(256), wfull((112, 128))],
            out_specs=tblk(128),
            out_shape=jax.ShapeDtypeStruct((Tp, 128), f32),
        )(g1, g2, heg, Wsbf[l])
        agg = jax.ops.segment_sum(msg, e2p, num_segments=E)
        if l == 0:
            he = pl.pallas_call(
                _update0_body,
                grid=(ge,),
                in_specs=[eblk(256), eblk(128), eblk(128),
                          wfull((128, 128)), wfull((1, 128))],
                out_specs=eblk(256),
                out_shape=jax.ShapeDtypeStruct((E, 256), f32),
            )(he, agg, ea1, Wlin[0], blin[0].reshape(1, 128))
        else:
            he = pl.pallas_call(
                _update1_body,
                grid=(ge,),
                in_specs=[eblk(256), eblk(128), wfull((128, 128)),
                          wfull((1, 128))],
                out_specs=eblk(128),
                out_shape=jax.ShapeDtypeStruct((E, 128), f32),
            )(he, agg, Wlin[1], blin[1].reshape(1, 128))

    isb = is_bond.astype(f32)[:, None]
    out = pl.pallas_call(
        functools.partial(_final_body, ge),
        grid=(ge,),
        in_specs=[eblk(128), eblk(1), pl.BlockSpec((8, 64), lambda i: (0, 0)),
                  wfull((64, 128)), wfull((128, 1)), wfull((1, 1))],
        out_specs=pl.BlockSpec((1, 1), lambda i: (0, 0)),
        out_shape=jax.ShapeDtypeStruct((1, 1), f32),
        scratch_shapes=[pltpu.VMEM((8, 128), f32)],
    )(he, isb, racc, Wr2, Wout, bout.reshape(1, 1))
    return out


# speedup vs baseline: 9.6910x; 1.2362x over previous
import functools

import jax
import jax.numpy as jnp
from jax import lax
from jax.experimental import pallas as pl
from jax.experimental.pallas import tpu as pltpu
from jax.experimental.pallas import tpu_sc as plsc

_CUT = 5.0
_PP = 5
_SBF = 7
_RBF = 16
_BE = 1600
_BT = 2048
_NW = 32
_CH = 128


def _silu(v):
    return v * jax.nn.sigmoid(v)


def _pack2(hi, lo):
    hb = jax.lax.bitcast_convert_type(hi, jnp.uint32)
    lb = jax.lax.bitcast_convert_type(lo, jnp.uint32)
    hb = (hb + jnp.uint32(0x8000)) >> 16
    lb = (lb + jnp.uint32(0x8000)) >> 16
    return jax.lax.bitcast_convert_type((hb << 16) | lb, jnp.int32)


def _unpack_hi(w):
    u = jax.lax.bitcast_convert_type(w, jnp.uint32)
    return jax.lax.bitcast_convert_type(u & jnp.uint32(0xFFFF0000),
                                        jnp.float32)


def _unpack_lo(w):
    u = jax.lax.bitcast_convert_type(w, jnp.uint32)
    return jax.lax.bitcast_convert_type(u << 16, jnp.float32)


def _env(d):
    xx = d / _CUT
    p = _PP
    e = (1.0 - ((p + 1) * (p + 2) / 2.0) * xx ** p
         + p * (p + 2) * xx ** (p + 1)
         - (p * (p + 1) / 2.0) * xx ** (p + 2))
    return jnp.where(d < _CUT, e, 0.0)


@functools.lru_cache(maxsize=None)
def _gather_fn(shape, dtype_name, M):
    dtype = jnp.dtype(dtype_name)
    per_w = M // _NW
    n_ch = per_w // _CH
    row_shape = tuple(shape[1:])
    mesh = plsc.VectorSubcoreMesh(core_axis_name="c", subcore_axis_name="s")

    @functools.partial(
        pl.kernel, mesh=mesh,
        out_type=jax.ShapeDtypeStruct((M,) + row_shape, dtype),
        scratch_types=[pltpu.VMEM((_CH,), jnp.int32),
                       pltpu.VMEM((_CH,) + row_shape, dtype),
                       pltpu.SemaphoreType.DMA],
    )
    def k(table_hbm, idx_hbm, out_hbm, idx_v, rows_v, sem):
        wid = lax.axis_index("s") * 2 + lax.axis_index("c")
        base = wid * per_w

        def body(i, carry):
            st = pl.multiple_of(base + i * _CH, _CH)
            pltpu.sync_copy(idx_hbm.at[pl.ds(st, _CH)], idx_v)
            pltpu.async_copy(table_hbm.at[idx_v], rows_v, sem).wait()
            pltpu.sync_copy(rows_v, out_hbm.at[pl.ds(st, _CH)])
            return carry

        lax.fori_loop(0, n_ch, body, 0)

    return k


def _sc_gather(table, idx):
    return _gather_fn(tuple(table.shape), table.dtype.name,
                      idx.shape[0])(table, idx)


def _edge_body(dv_ref, eattr_ref, eae_ref, wmat_ref, bmat_ref,
               wemb_ref, bemb_ref, wrbf_ref, brbf_ref, wea0_ref, wea1_ref,
               he0_ref, geom_ref, ea1_ref, racc_ref):
    i = pl.program_id(0)
    blk = dv_ref.shape[0]
    dv = dv_ref[...]
    d = jnp.sqrt(jnp.sum(dv * dv, axis=1, keepdims=True) + 1e-12)
    env = _env(d)
    ea = eattr_ref[...] * env
    t1 = _silu(jnp.dot(ea, wmat_ref[...], preferred_element_type=jnp.float32)
               + bmat_ref[...])
    h0 = _silu(jnp.dot(t1, wemb_ref[...], preferred_element_type=jnp.float32)
               + bemb_ref[...])
    n = (1 + jax.lax.broadcasted_iota(jnp.int32, (blk, _RBF), 1)
         ).astype(jnp.float32)
    rb = (jnp.sqrt(2.0 / _CUT) * jnp.sin(n * (jnp.pi / _CUT) * d)
          / (d + 1e-9)) * env
    geom_ref[...] = jnp.concatenate(
        [dv, rb, jnp.zeros((blk, 128 - 19), jnp.float32)], axis=1)
    eae = eae_ref[...]
    ea0 = _silu(jnp.dot(eae, wea0_ref[...],
                        preferred_element_type=jnp.float32))
    ea1_ref[...] = _silu(jnp.dot(eae, wea1_ref[...],
                                 preferred_element_type=jnp.float32))
    he0_ref[...] = _pack2(h0, ea0)
    rbf64 = _silu(jnp.dot(rb, wrbf_ref[...],
                          preferred_element_type=jnp.float32) + brbf_ref[...])
    colsum = jnp.sum(rbf64, axis=0, keepdims=True)

    @pl.when(i == 0)
    def _():
        racc_ref[...] = jnp.zeros_like(racc_ref)

    row0 = jax.lax.broadcasted_iota(jnp.int32, (8, 64), 0) == 0
    racc_ref[...] += jnp.where(row0, jnp.broadcast_to(colsum, (8, 64)), 0.0)


def _tstat_body(g1_ref, g2_ref, ts_ref):
    blk = g1_ref.shape[0]
    dv1 = g1_ref[:, 0:3]
    dv2 = g2_ref[:, 0:3]
    rbg = g1_ref[:, 3:19]
    cos_a = -jnp.sum(dv1 * dv2, axis=1, keepdims=True)
    c0 = dv1[:, 1:2] * dv2[:, 2:3] - dv1[:, 2:3] * dv2[:, 1:2]
    c1 = dv1[:, 2:3] * dv2[:, 0:1] - dv1[:, 0:1] * dv2[:, 2:3]
    c2 = dv1[:, 0:1] * dv2[:, 1:2] - dv1[:, 1:2] * dv2[:, 0:1]
    sin2 = c0 * c0 + c1 * c1 + c2 * c2
    hyp = jnp.sqrt(cos_a * cos_a + sin2)
    cost = jnp.where(hyp > 1e-30, cos_a / jnp.maximum(hyp, 1e-30), 1.0)
    ts_ref[...] = jnp.concatenate(
        [cost, rbg, jnp.zeros((blk, 15), jnp.float32)], axis=1)


def _triplet_body(n_real, ts_ref, he_ref, wsbf_ref, msg_ref):
    i = pl.program_id(0)
    blk = ts_ref.shape[0]
    pos = i * blk + jax.lax.broadcasted_iota(jnp.int32, (blk, 1), 0)
    cost = ts_ref[:, 0:1]
    rbg = ts_ref[:, 1:17]
    angs = [jnp.ones_like(cost), cost]
    for _ in range(2, _SBF):
        angs.append(2.0 * cost * angs[-1] - angs[-2])
    sbf = jnp.concatenate([a * rbg for a in angs], axis=1)
    gate = _silu(jnp.dot(sbf, wsbf_ref[...],
                         preferred_element_type=jnp.float32))
    w = he_ref[...]
    msg = _unpack_hi(w) * gate + _unpack_lo(w)
    msg_ref[...] = jnp.where(pos < n_real, msg, 0.0).astype(jnp.bfloat16)


def _update0_body(he_ref, agg_ref, ea1_ref, wlin_ref, blin_ref, out_ref):
    h0 = _unpack_hi(he_ref[...])
    agg = agg_ref[...].astype(jnp.float32)
    h1 = _silu(jnp.dot(h0 + agg, wlin_ref[...],
                       preferred_element_type=jnp.float32) + blin_ref[...])
    out_ref[...] = _pack2(h1, ea1_ref[...])


def _update1_body(he_ref, agg_ref, wlin_ref, blin_ref, out_ref):
    h1 = _unpack_hi(he_ref[...])
    agg = agg_ref[...].astype(jnp.float32)
    out_ref[...] = _silu(
        jnp.dot(h1 + agg, wlin_ref[...],
                preferred_element_type=jnp.float32) + blin_ref[...])


def _final_body(nblk, h_ref, isb_ref, racc_ref, wr2_ref, wout_ref,
                bout_ref, out_ref, acc_ref):
    i = pl.program_id(0)
    wb = jnp.where(isb_ref[...] > 0.5, 1.0, 0.5)
    hw = h_ref[...] * wb
    colsum = jnp.sum(hw, axis=0, keepdims=True)

    @pl.when(i == 0)
    def _():
        acc_ref[...] = jnp.zeros_like(acc_ref)

    row0 = jax.lax.broadcasted_iota(jnp.int32, (8, 128), 0) == 0
    acc_ref[...] += jnp.where(row0, jnp.broadcast_to(colsum, (8, 128)), 0.0)

    @pl.when(i == nblk - 1)
    def _():
        rsum = racc_ref[0:1, :]
        total = acc_ref[0:1, :] + jnp.dot(
            rsum, wr2_ref[...], preferred_element_type=jnp.float32)
        out_ref[...] = (jnp.dot(total, wout_ref[...],
                                preferred_element_type=jnp.float32)
                        + bout_ref[...])


def _pad_to(a, n):
    return a if a.shape[0] == n else jnp.pad(a, (0, n - a.shape[0]))


def kernel(atom_pos, edge_attr, x, edge_index, is_bond, neo_edge_index,
           atom_j, atom_i, atom_k, emb_table, W_mat, b_mat, W_rbf, b_rbf,
           W_emb, b_emb, Wsbf, Wea, Wlin, blin, Wr2, Wout, bout):
    f32 = jnp.float32
    bf16 = jnp.bfloat16
    src = edge_index[0].astype(jnp.int32)
    dst = edge_index[1].astype(jnp.int32)
    e1 = neo_edge_index[0].astype(jnp.int32)
    e2 = neo_edge_index[1].astype(jnp.int32)
    E = edge_attr.shape[0]
    T = e1.shape[0]
    unit = _NW * _CH
    Tp = ((T + unit - 1) // unit) * unit
    ge = E // _BE
    gt = Tp // _BT

    e1p = _pad_to(e1, Tp)
    e2p = _pad_to(e2, Tp)

    dvec = atom_pos[src] - atom_pos[dst]
    eae = emb_table[x][dst]

    wfull = lambda shp: pl.BlockSpec(shp, lambda i: (0, 0))
    eblk = lambda c: pl.BlockSpec((_BE, c), lambda i: (i, 0))
    tblk = lambda c: pl.BlockSpec((_BT, c), lambda i: (i, 0))

    he0, geom, ea1, racc = pl.pallas_call(
        _edge_body,
        grid=(ge,),
        in_specs=[eblk(3), eblk(338), eblk(64), wfull((338, 128)),
                  wfull((1, 128)), wfull((128, 128)), wfull((1, 128)),
                  wfull((16, 64)), wfull((1, 64)), wfull((64, 128)),
                  wfull((64, 128))],
        out_specs=[eblk(128), eblk(128), eblk(128),
                   pl.BlockSpec((8, 64), lambda i: (0, 0))],
        out_shape=[jax.ShapeDtypeStruct((E, 128), jnp.int32),
                   jax.ShapeDtypeStruct((E, 128), f32),
                   jax.ShapeDtypeStruct((E, 128), f32),
                   jax.ShapeDtypeStruct((8, 64), f32)],
    )(dvec, edge_attr, eae, W_mat, b_mat.reshape(1, 128),
      W_emb, b_emb.reshape(1, 128), W_rbf, b_rbf.reshape(1, 64),
      Wea[0], Wea[1])

    g1 = _sc_gather(geom, e1p)
    g2 = _sc_gather(geom, e2p)
    tstat = pl.pallas_call(
        _tstat_body,
        grid=(gt,),
        in_specs=[tblk(128), tblk(128)],
        out_specs=tblk(32),
        out_shape=jax.ShapeDtypeStruct((Tp, 32), f32),
    )(g1, g2)

    he = he0
    for l in range(Wlin.shape[0]):
        heg = _sc_gather(he, e1p)
        msg = pl.pallas_call(
            functools.partial(_triplet_body, T),
            grid=(gt,),
            in_specs=[tblk(32), tblk(128), wfull((112, 128))],
            out_specs=tblk(128),
            out_shape=jax.ShapeDtypeStruct((Tp, 128), bf16),
        )(tstat, heg, Wsbf[l])
        agg = jax.ops.segment_sum(msg, e2p, num_segments=E)
        if l == 0:
            he = pl.pallas_call(
                _update0_body,
                grid=(ge,),
                in_specs=[eblk(128), eblk(128), eblk(128),
                          wfull((128, 128)), wfull((1, 128))],
                out_specs=eblk(128),
                out_shape=jax.ShapeDtypeStruct((E, 128), jnp.int32),
            )(he, agg, ea1, Wlin[0], blin[0].reshape(1, 128))
        else:
            he = pl.pallas_call(
                _update1_body,
                grid=(ge,),
                in_specs=[eblk(128), eblk(128), wfull((128, 128)),
                          wfull((1, 128))],
                out_specs=eblk(128),
                out_shape=jax.ShapeDtypeStruct((E, 128), f32),
            )(he, agg, Wlin[1], blin[1].reshape(1, 128))

    isb = is_bond.astype(f32)[:, None]
    out = pl.pallas_call(
        functools.partial(_final_body, ge),
        grid=(ge,),
        in_specs=[eblk(128), eblk(1), pl.BlockSpec((8, 64), lambda i: (0, 0)),
                  wfull((64, 128)), wfull((128, 1)), wfull((1, 1))],
        out_specs=pl.BlockSpec((1, 1), lambda i: (0, 0)),
        out_shape=jax.ShapeDtypeStruct((1, 1), f32),
        scratch_shapes=[pltpu.VMEM((8, 128), f32)],
    )(he, isb, racc, Wr2, Wout, bout.reshape(1, 1))
    return out


# speedup vs baseline: 9.9339x; 1.0251x over previous
import functools

import jax
import jax.numpy as jnp
from jax import lax
from jax.experimental import pallas as pl
from jax.experimental.pallas import tpu as pltpu
from jax.experimental.pallas import tpu_sc as plsc

_CUT = 5.0
_PP = 5
_SBF = 7
_RBF = 16
_BE = 1600
_BT = 2048
_NW = 32
_CH = 128


def _silu(v):
    return v * jax.nn.sigmoid(v)


def _pack2(hi, lo):
    hb = jax.lax.bitcast_convert_type(hi, jnp.uint32)
    lb = jax.lax.bitcast_convert_type(lo, jnp.uint32)
    hb = (hb + jnp.uint32(0x8000)) >> 16
    lb = (lb + jnp.uint32(0x8000)) >> 16
    return jax.lax.bitcast_convert_type((hb << 16) | lb, jnp.int32)


def _unpack_hi(w):
    u = jax.lax.bitcast_convert_type(w, jnp.uint32)
    return jax.lax.bitcast_convert_type(u & jnp.uint32(0xFFFF0000),
                                        jnp.float32)


def _unpack_lo(w):
    u = jax.lax.bitcast_convert_type(w, jnp.uint32)
    return jax.lax.bitcast_convert_type(u << 16, jnp.float32)


def _env(d):
    xx = d / _CUT
    p = _PP
    e = (1.0 - ((p + 1) * (p + 2) / 2.0) * xx ** p
         + p * (p + 2) * xx ** (p + 1)
         - (p * (p + 1) / 2.0) * xx ** (p + 2))
    return jnp.where(d < _CUT, e, 0.0)


_KB = 4


@functools.lru_cache(maxsize=None)
def _gather_fn(shape, dtype_name, M):
    dtype = jnp.dtype(dtype_name)
    per_w = M // _NW
    blk = _KB * _CH
    n_blk = per_w // blk
    D = shape[1]
    mesh = plsc.VectorSubcoreMesh(core_axis_name="c", subcore_axis_name="s")

    @functools.partial(
        pl.kernel, mesh=mesh,
        out_type=jax.ShapeDtypeStruct((M, D), dtype),
        scratch_types=[pltpu.VMEM((blk,), jnp.int32),
                       pltpu.VMEM((blk, D), dtype),
                       pltpu.SemaphoreType.DMA],
    )
    def k(table_hbm, idx_hbm, out_hbm, idx_v, rows_v, sem):
        wid = lax.axis_index("s") * 2 + lax.axis_index("c")
        base = wid * per_w

        def body(i, carry):
            st = pl.multiple_of(base + i * blk, blk)
            pltpu.sync_copy(idx_hbm.at[pl.ds(st, blk)], idx_v)
            copies = [
                pltpu.async_copy(
                    table_hbm.at[idx_v.at[pl.ds(kk * _CH, _CH)]],
                    rows_v.at[pl.ds(kk * _CH, _CH)], sem)
                for kk in range(_KB)
            ]
            for c in copies:
                c.wait()
            pltpu.sync_copy(rows_v, out_hbm.at[pl.ds(st, blk)])
            return carry

        lax.fori_loop(0, n_blk, body, 0)

    return k


def _sc_gather(table, idx):
    return _gather_fn(tuple(table.shape), table.dtype.name,
                      idx.shape[0])(table, idx)


def _edge_body(dv_ref, eattr_ref, eae_ref, wmat_ref, bmat_ref,
               wemb_ref, bemb_ref, wrbf_ref, brbf_ref, wea0_ref, wea1_ref,
               he0_ref, geom_ref, ea1_ref, racc_ref):
    i = pl.program_id(0)
    blk = dv_ref.shape[0]
    dv = dv_ref[...]
    d = jnp.sqrt(jnp.sum(dv * dv, axis=1, keepdims=True) + 1e-12)
    env = _env(d)
    ea = eattr_ref[...] * env
    t1 = _silu(jnp.dot(ea, wmat_ref[...], preferred_element_type=jnp.float32)
               + bmat_ref[...])
    h0 = _silu(jnp.dot(t1, wemb_ref[...], preferred_element_type=jnp.float32)
               + bemb_ref[...])
    n = (1 + jax.lax.broadcasted_iota(jnp.int32, (blk, _RBF), 1)
         ).astype(jnp.float32)
    rb = (jnp.sqrt(2.0 / _CUT) * jnp.sin(n * (jnp.pi / _CUT) * d)
          / (d + 1e-9)) * env
    geom_ref[...] = jnp.concatenate(
        [dv, rb, jnp.zeros((blk, 128 - 19), jnp.float32)], axis=1)
    eae = eae_ref[...]
    ea0 = _silu(jnp.dot(eae, wea0_ref[...],
                        preferred_element_type=jnp.float32))
    ea1_ref[...] = _silu(jnp.dot(eae, wea1_ref[...],
                                 preferred_element_type=jnp.float32))
    he0_ref[...] = _pack2(h0, ea0)
    rbf64 = _silu(jnp.dot(rb, wrbf_ref[...],
                          preferred_element_type=jnp.float32) + brbf_ref[...])
    colsum = jnp.sum(rbf64, axis=0, keepdims=True)

    @pl.when(i == 0)
    def _():
        racc_ref[...] = jnp.zeros_like(racc_ref)

    row0 = jax.lax.broadcasted_iota(jnp.int32, (8, 64), 0) == 0
    racc_ref[...] += jnp.where(row0, jnp.broadcast_to(colsum, (8, 64)), 0.0)


def _tstat_body(g1_ref, g2_ref, ts_ref):
    blk = g1_ref.shape[0]
    dv1 = g1_ref[:, 0:3]
    dv2 = g2_ref[:, 0:3]
    rbg = g1_ref[:, 3:19]
    cos_a = -jnp.sum(dv1 * dv2, axis=1, keepdims=True)
    c0 = dv1[:, 1:2] * dv2[:, 2:3] - dv1[:, 2:3] * dv2[:, 1:2]
    c1 = dv1[:, 2:3] * dv2[:, 0:1] - dv1[:, 0:1] * dv2[:, 2:3]
    c2 = dv1[:, 0:1] * dv2[:, 1:2] - dv1[:, 1:2] * dv2[:, 0:1]
    sin2 = c0 * c0 + c1 * c1 + c2 * c2
    hyp = jnp.sqrt(cos_a * cos_a + sin2)
    cost = jnp.where(hyp > 1e-30, cos_a / jnp.maximum(hyp, 1e-30), 1.0)
    ts_ref[...] = jnp.concatenate(
        [cost, rbg, jnp.zeros((blk, 15), jnp.float32)], axis=1)


def _triplet_body(n_real, ts_ref, he_ref, wsbf_ref, msg_ref):
    i = pl.program_id(0)
    blk = ts_ref.shape[0]
    pos = i * blk + jax.lax.broadcasted_iota(jnp.int32, (blk, 1), 0)
    cost = ts_ref[:, 0:1]
    rbg = ts_ref[:, 1:17]
    angs = [jnp.ones_like(cost), cost]
    for _ in range(2, _SBF):
        angs.append(2.0 * cost * angs[-1] - angs[-2])
    sbf = jnp.concatenate([a * rbg for a in angs], axis=1)
    gate = _silu(jnp.dot(sbf, wsbf_ref[...],
                         preferred_element_type=jnp.float32))
    w = he_ref[...]
    msg = _unpack_hi(w) * gate + _unpack_lo(w)
    msg_ref[...] = jnp.where(pos < n_real, msg, 0.0).astype(jnp.bfloat16)


def _update0_body(he_ref, agg_ref, ea1_ref, wlin_ref, blin_ref, out_ref):
    h0 = _unpack_hi(he_ref[...])
    agg = agg_ref[...].astype(jnp.float32)
    h1 = _silu(jnp.dot(h0 + agg, wlin_ref[...],
                       preferred_element_type=jnp.float32) + blin_ref[...])
    out_ref[...] = _pack2(h1, ea1_ref[...])


def _update1_body(he_ref, agg_ref, wlin_ref, blin_ref, out_ref):
    h1 = _unpack_hi(he_ref[...])
    agg = agg_ref[...].astype(jnp.float32)
    out_ref[...] = _silu(
        jnp.dot(h1 + agg, wlin_ref[...],
                preferred_element_type=jnp.float32) + blin_ref[...])


def _final_body(nblk, h_ref, isb_ref, racc_ref, wr2_ref, wout_ref,
                bout_ref, out_ref, acc_ref):
    i = pl.program_id(0)
    wb = jnp.where(isb_ref[...] > 0.5, 1.0, 0.5)
    hw = h_ref[...] * wb
    colsum = jnp.sum(hw, axis=0, keepdims=True)

    @pl.when(i == 0)
    def _():
        acc_ref[...] = jnp.zeros_like(acc_ref)

    row0 = jax.lax.broadcasted_iota(jnp.int32, (8, 128), 0) == 0
    acc_ref[...] += jnp.where(row0, jnp.broadcast_to(colsum, (8, 128)), 0.0)

    @pl.when(i == nblk - 1)
    def _():
        rsum = racc_ref[0:1, :]
        total = acc_ref[0:1, :] + jnp.dot(
            rsum, wr2_ref[...], preferred_element_type=jnp.float32)
        out_ref[...] = (jnp.dot(total, wout_ref[...],
                                preferred_element_type=jnp.float32)
                        + bout_ref[...])


def _pad_to(a, n):
    return a if a.shape[0] == n else jnp.pad(a, (0, n - a.shape[0]))


def kernel(atom_pos, edge_attr, x, edge_index, is_bond, neo_edge_index,
           atom_j, atom_i, atom_k, emb_table, W_mat, b_mat, W_rbf, b_rbf,
           W_emb, b_emb, Wsbf, Wea, Wlin, blin, Wr2, Wout, bout):
    f32 = jnp.float32
    bf16 = jnp.bfloat16
    src = edge_index[0].astype(jnp.int32)
    dst = edge_index[1].astype(jnp.int32)
    e1 = neo_edge_index[0].astype(jnp.int32)
    e2 = neo_edge_index[1].astype(jnp.int32)
    E = edge_attr.shape[0]
    T = e1.shape[0]
    unit = _NW * _KB * _CH
    Tp = ((T + unit - 1) // unit) * unit
    ge = E // _BE
    gt = Tp // _BT

    e1p = _pad_to(e1, Tp)
    e2p = _pad_to(e2, Tp)

    dvec = atom_pos[src] - atom_pos[dst]
    eae = emb_table[x][dst]

    wfull = lambda shp: pl.BlockSpec(shp, lambda i: (0, 0))
    eblk = lambda c: pl.BlockSpec((_BE, c), lambda i: (i, 0))
    tblk = lambda c: pl.BlockSpec((_BT, c), lambda i: (i, 0))

    he0, geom, ea1, racc = pl.pallas_call(
        _edge_body,
        grid=(ge,),
        in_specs=[eblk(3), eblk(338), eblk(64), wfull((338, 128)),
                  wfull((1, 128)), wfull((128, 128)), wfull((1, 128)),
                  wfull((16, 64)), wfull((1, 64)), wfull((64, 128)),
                  wfull((64, 128))],
        out_specs=[eblk(128), eblk(128), eblk(128),
                   pl.BlockSpec((8, 64), lambda i: (0, 0))],
        out_shape=[jax.ShapeDtypeStruct((E, 128), jnp.int32),
                   jax.ShapeDtypeStruct((E, 128), f32),
                   jax.ShapeDtypeStruct((E, 128), f32),
                   jax.ShapeDtypeStruct((8, 64), f32)],
    )(dvec, edge_attr, eae, W_mat, b_mat.reshape(1, 128),
      W_emb, b_emb.reshape(1, 128), W_rbf, b_rbf.reshape(1, 64),
      Wea[0], Wea[1])

    g1 = _sc_gather(geom, e1p)
    g2 = _sc_gather(geom, e2p)
    tstat = pl.pallas_call(
        _tstat_body,
        grid=(gt,),
        in_specs=[tblk(128), tblk(128)],
        out_specs=tblk(32),
        out_shape=jax.ShapeDtypeStruct((Tp, 32), f32),
    )(g1, g2)

    he = he0
    for l in range(Wlin.shape[0]):
        heg = _sc_gather(he, e1p)
        msg = pl.pallas_call(
            functools.partial(_triplet_body, T),
            grid=(gt,),
            in_specs=[tblk(32), tblk(128), wfull((112, 128))],
            out_specs=tblk(128),
            out_shape=jax.ShapeDtypeStruct((Tp, 128), bf16),
        )(tstat, heg, Wsbf[l])
        agg = jax.ops.segment_sum(msg, e2p, num_segments=E)
        if l == 0:
            he = pl.pallas_call(
                _update0_body,
                grid=(ge,),
                in_specs=[eblk(128), eblk(128), eblk(128),
                          wfull((128, 128)), wfull((1, 128))],
                out_specs=eblk(128),
                out_shape=jax.ShapeDtypeStruct((E, 128), jnp.int32),
            )(he, agg, ea1, Wlin[0], blin[0].reshape(1, 128))
        else:
            he = pl.pallas_call(
                _update1_body,
                grid=(ge,),
                in_specs=[eblk(128), eblk(128), wfull((128, 128)),
                          wfull((1, 128))],
                out_specs=eblk(128),
                out_shape=jax.ShapeDtypeStruct((E, 128), f32),
            )(he, agg, Wlin[1], blin[1].reshape(1, 128))

    isb = is_bond.astype(f32)[:, None]
    out = pl.pallas_call(
        functools.partial(_final_body, ge),
        grid=(ge,),
        in_specs=[eblk(128), eblk(1), pl.BlockSpec((8, 64), lambda i: (0, 0)),
                  wfull((64, 128)), wfull((128, 1)), wfull((1, 1))],
        out_specs=pl.BlockSpec((1, 1), lambda i: (0, 0)),
        out_shape=jax.ShapeDtypeStruct((1, 1), f32),
        scratch_shapes=[pltpu.VMEM((8, 128), f32)],
    )(he, isb, racc, Wr2, Wout, bout.reshape(1, 1))
    return out
